# baseline (speedup 1.0000x reference)
import functools
import math

import jax
import jax.numpy as jnp
from jax import lax
from jax.experimental import pallas as pl
from jax.experimental.pallas import tpu as pltpu
from jax.experimental.pallas import tpu_sc as plsc

_N = 10000
_D = 128
_LANE = 128
_NC = 2
_NS = 16
_NW = _NC * _NS

_ACC_ROWS = 10112
_APT = _ACC_ROWS // _NS
_CAC = 10240
_CPT = _CAC // _NS
_G = 8


def _make_sc_agg(e_pad, with_cnt):
  iters_per_tile = e_pad // (_NW * _LANE)
  mesh = plsc.VectorSubcoreMesh(core_axis_name="c", subcore_axis_name="s")

  out_type = [jax.ShapeDtypeStruct((_NC * _ACC_ROWS, _D), jnp.float32)]
  scratch = [
      pltpu.VMEM_SHARED((_ACC_ROWS, _D), jnp.float32),
      pltpu.VMEM((_G, _LANE), jnp.int32),
      pltpu.VMEM((_G, _LANE), jnp.int32),
      pltpu.VMEM((_LANE, _D), jnp.float32),
      pltpu.SemaphoreType.DMA,
  ]
  if with_cnt:
    out_type.append(jax.ShapeDtypeStruct((_NC * _CAC,), jnp.float32))
    scratch += [
        pltpu.VMEM_SHARED((_CAC,), jnp.float32),
        pltpu.VMEM((_LANE,), jnp.float32),
        pltpu.VMEM((_CPT,), jnp.float32),
    ]

  def body(*refs):
    if with_cnt:
      (table, srcp, dstp, zrow, zcnt,
       out, cntout, acc, srcv, dstv, rows, gsem, cacc, ones_v, cbuf) = refs
    else:
      (table, srcp, dstp, zrow,
       out, acc, srcv, dstv, rows, gsem) = refs
    c = lax.axis_index("c")
    s = lax.axis_index("s")
    r0 = s * _APT
    pltpu.sync_copy(zrow, acc.at[pl.ds(r0, _APT)])
    if with_cnt:
      c0 = s * _CPT
      for j in range(_LANE // 16):
        ones_v[pl.ds(j * 16, 16)] = jnp.ones((16,), jnp.float32)
      pltpu.sync_copy(zcnt, cbuf)
      pltpu.sync_copy(cbuf, cacc.at[pl.ds(c0, _CPT)])
    plsc.subcore_barrier()

    base_row = (c * _NS + s) * iters_per_tile

    def group(g, carry):
      grow = base_row + g * _G
      pltpu.sync_copy(srcp.at[pl.ds(grow, _G)], srcv)
      pltpu.sync_copy(dstp.at[pl.ds(grow, _G)], dstv)
      for j in range(_G):
        pltpu.async_copy(table.at[srcv.at[j]], rows, gsem).wait()
        pltpu.sync_copy(rows, acc.at[dstv.at[j]], add=True)
        if with_cnt:
          pltpu.sync_copy(ones_v, cacc.at[dstv.at[j]], add=True)
      return carry

    lax.fori_loop(0, iters_per_tile // _G, group, 0)
    plsc.subcore_barrier()
    pltpu.sync_copy(acc.at[pl.ds(r0, _APT)], out.at[pl.ds(c * _ACC_ROWS + r0, _APT)])
    if with_cnt:
      pltpu.sync_copy(cacc.at[pl.ds(c0, _CPT)], cbuf)
      pltpu.sync_copy(cbuf, cntout.at[pl.ds(c * _CAC + c0, _CPT)])

  return pl.kernel(body, out_type=out_type, mesh=mesh, scratch_types=scratch)


def _dot_t(a, w):
  return lax.dot_general(a, w, (((1,), (1,)), ((), ())),
                         precision=lax.Precision.HIGHEST,
                         preferred_element_type=jnp.float32)


_R = 2000


def _row_spec(shape3=None, shape2=None):
  if shape3 is not None:
    return pl.BlockSpec((2, _R) + shape3, lambda i: (0, i, 0))
  return pl.BlockSpec((_R,) + shape2, lambda i: (i, 0))


def _full_spec(shape):
  nd = len(shape)
  return pl.BlockSpec(shape, lambda i, _n=nd: (0,) * _n)


def _tc_layer1(sum2, cnt2, x, W1l, b1, W1r):
  def body(s_ref, c_ref, x_ref, wl_ref, b_ref, wr_ref, h_ref):
    cnt = jnp.maximum(c_ref[0] + c_ref[1], 1.0)
    mean = (s_ref[0] + s_ref[1]) / cnt
    h = _dot_t(mean, wl_ref[...]) + b_ref[...] + _dot_t(x_ref[...], wr_ref[...])
    h_ref[...] = jnp.maximum(h, 0.0)

  return pl.pallas_call(
      body,
      grid=(_N // _R,),
      in_specs=[_row_spec(shape3=(_D,)), _row_spec(shape3=(1,)),
                _row_spec(shape2=(_D,)), _full_spec((_D, _D)),
                _full_spec((1, _D)), _full_spec((_D, _D))],
      out_specs=_row_spec(shape2=(_D,)),
      out_shape=jax.ShapeDtypeStruct((_N, _D), jnp.float32),
  )(sum2, cnt2, x, W1l, b1, W1r)


def _tc_layer23(sum2, cnt2, h, Wmul, bmu, Wmur, Wlsl, bls, Wlsr):
  def body(s_ref, c_ref, h_ref, wmul_ref, bmu_ref, wmur_ref,
           wlsl_ref, bls_ref, wlsr_ref, mu_ref, ls_ref):
    cnt = jnp.maximum(c_ref[0] + c_ref[1], 1.0)
    mean = (s_ref[0] + s_ref[1]) / cnt
    hh = h_ref[...]
    mu_ref[...] = _dot_t(mean, wmul_ref[...]) + bmu_ref[...] + _dot_t(hh, wmur_ref[...])
    ls_ref[...] = _dot_t(mean, wlsl_ref[...]) + bls_ref[...] + _dot_t(hh, wlsr_ref[...])

  return pl.pallas_call(
      body,
      grid=(_N // _R,),
      in_specs=[_row_spec(shape3=(_D,)), _row_spec(shape3=(1,)),
                _row_spec(shape2=(_D,)),
                _full_spec((64, _D)), _full_spec((1, 64)), _full_spec((64, _D)),
                _full_spec((64, _D)), _full_spec((1, 64)), _full_spec((64, _D))],
      out_specs=[_row_spec(shape2=(64,)), _row_spec(shape2=(64,))],
      out_shape=[jax.ShapeDtypeStruct((_N, 64), jnp.float32),
                 jax.ShapeDtypeStruct((_N, 64), jnp.float32)],
  )(sum2, cnt2, h, Wmul, bmu, Wmur, Wlsl, bls, Wlsr)


def kernel(x, edge_index, W1l, b1, W1r, Wmul, bmu, Wmur, Wlsl, bls, Wlsr):
  e = edge_index.shape[1]
  e_pad = math.ceil(e / (_LANE * _NW * _G)) * _LANE * _NW * _G
  pad = e_pad - e
  src = jnp.concatenate(
      [edge_index[0], jnp.zeros((pad,), jnp.int32)]).reshape(-1, _LANE)
  dst = jnp.concatenate(
      [edge_index[1], jnp.full((pad,), _N, jnp.int32)]).reshape(-1, _LANE)
  zrow = jnp.zeros((_APT, _D), jnp.float32)
  zcnt = jnp.zeros((_CPT,), jnp.float32)

  agg_cnt = _make_sc_agg(e_pad, with_cnt=True)
  agg = _make_sc_agg(e_pad, with_cnt=False)

  sum_flat, cnt_flat = agg_cnt(x, src, dst, zrow, zcnt)
  sum2 = sum_flat.reshape(_NC, _ACC_ROWS, _D)[:, :_N]
  cnt2 = cnt_flat.reshape(_NC, _CAC)[:, :_N].reshape(_NC, _N, 1)
  h = _tc_layer1(sum2, cnt2, x, W1l, b1.reshape(1, _D), W1r)

  (sumh_flat,) = agg(h, src, dst, zrow)
  sumh2 = sumh_flat.reshape(_NC, _ACC_ROWS, _D)[:, :_N]
  mu, logstd = _tc_layer23(sumh2, cnt2, h, Wmul, bmu.reshape(1, 64), Wmur,
                           Wlsl, bls.reshape(1, 64), Wlsr)
  return (mu, logstd)


# speedup vs baseline: 1.2622x; 1.2622x over previous
import functools
import math

import jax
import jax.numpy as jnp
from jax import lax
from jax.experimental import pallas as pl
from jax.experimental.pallas import tpu as pltpu
from jax.experimental.pallas import tpu_sc as plsc

_N = 10000
_D = 128
_LANE = 128
_NC = 2
_NS = 16
_NW = _NC * _NS

_ACC_ROWS = 10112
_APT = _ACC_ROWS // _NS
_CAC = 10240
_CPT = _CAC // _NS


def _make_sc_agg(e_pad, with_cnt):
  iters_per_tile = e_pad // (_NW * _LANE)
  mesh = plsc.VectorSubcoreMesh(core_axis_name="c", subcore_axis_name="s")

  out_type = [jax.ShapeDtypeStruct((_NC * _ACC_ROWS, _D), jnp.float32)]
  scratch = [
      pltpu.VMEM_SHARED((_ACC_ROWS, _D), jnp.float32),
      pltpu.VMEM((_LANE,), jnp.int32),
      pltpu.VMEM((_LANE,), jnp.int32),
      pltpu.VMEM((_LANE, _D), jnp.float32),
      pltpu.SemaphoreType.DMA,
  ]
  if with_cnt:
    out_type.append(jax.ShapeDtypeStruct((_NC * _CAC,), jnp.float32))
    scratch += [
        pltpu.VMEM_SHARED((_CAC,), jnp.float32),
        pltpu.VMEM((_LANE,), jnp.float32),
        pltpu.VMEM((_CPT,), jnp.float32),
    ]

  def body(*refs):
    if with_cnt:
      (table, srcp, dstp, zrow, zcnt,
       out, cntout, acc, srcv, dstv, rows, gsem, cacc, ones_v, cbuf) = refs
    else:
      (table, srcp, dstp, zrow,
       out, acc, srcv, dstv, rows, gsem) = refs
    c = lax.axis_index("c")
    s = lax.axis_index("s")
    r0 = s * _APT
    pltpu.sync_copy(zrow, acc.at[pl.ds(r0, _APT)])
    if with_cnt:
      c0 = s * _CPT
      for j in range(_LANE // 16):
        ones_v[pl.ds(j * 16, 16)] = jnp.ones((16,), jnp.float32)
      pltpu.sync_copy(zcnt, cbuf)
      pltpu.sync_copy(cbuf, cacc.at[pl.ds(c0, _CPT)])
    plsc.subcore_barrier()

    base = (c * _NS + s) * iters_per_tile * _LANE

    def step(i, carry):
      off = base + i * _LANE
      pltpu.sync_copy(srcp.at[pl.ds(off, _LANE)], srcv)
      pltpu.sync_copy(dstp.at[pl.ds(off, _LANE)], dstv)
      pltpu.async_copy(table.at[srcv], rows, gsem).wait()
      pltpu.sync_copy(rows, acc.at[dstv], add=True)
      if with_cnt:
        pltpu.sync_copy(ones_v, cacc.at[dstv], add=True)
      return carry

    lax.fori_loop(0, iters_per_tile, step, 0)
    plsc.subcore_barrier()
    pltpu.sync_copy(acc.at[pl.ds(r0, _APT)], out.at[pl.ds(c * _ACC_ROWS + r0, _APT)])
    if with_cnt:
      pltpu.sync_copy(cacc.at[pl.ds(c0, _CPT)], cbuf)
      pltpu.sync_copy(cbuf, cntout.at[pl.ds(c * _CAC + c0, _CPT)])

  return pl.kernel(body, out_type=out_type, mesh=mesh, scratch_types=scratch)


def _dot_t(a, w):
  return lax.dot_general(a, w, (((1,), (1,)), ((), ())),
                         precision=lax.Precision.HIGHEST,
                         preferred_element_type=jnp.float32)


_R = 2000


def _row_spec(shape3=None, shape2=None):
  if shape3 is not None:
    return pl.BlockSpec((2, _R) + shape3, lambda i: (0, i, 0))
  return pl.BlockSpec((_R,) + shape2, lambda i: (i, 0))


def _full_spec(shape):
  nd = len(shape)
  return pl.BlockSpec(shape, lambda i, _n=nd: (0,) * _n)


def _tc_layer1(sum2, cnt2, x, W1l, b1, W1r):
  def body(s_ref, c_ref, x_ref, wl_ref, b_ref, wr_ref, h_ref):
    cnt = jnp.maximum(c_ref[0] + c_ref[1], 1.0)
    mean = (s_ref[0] + s_ref[1]) / cnt
    h = _dot_t(mean, wl_ref[...]) + b_ref[...] + _dot_t(x_ref[...], wr_ref[...])
    h_ref[...] = jnp.maximum(h, 0.0)

  return pl.pallas_call(
      body,
      grid=(_N // _R,),
      in_specs=[_row_spec(shape3=(_D,)), _row_spec(shape3=(1,)),
                _row_spec(shape2=(_D,)), _full_spec((_D, _D)),
                _full_spec((1, _D)), _full_spec((_D, _D))],
      out_specs=_row_spec(shape2=(_D,)),
      out_shape=jax.ShapeDtypeStruct((_N, _D), jnp.float32),
  )(sum2, cnt2, x, W1l, b1, W1r)


def _tc_layer23(sum2, cnt2, h, Wmul, bmu, Wmur, Wlsl, bls, Wlsr):
  def body(s_ref, c_ref, h_ref, wmul_ref, bmu_ref, wmur_ref,
           wlsl_ref, bls_ref, wlsr_ref, mu_ref, ls_ref):
    cnt = jnp.maximum(c_ref[0] + c_ref[1], 1.0)
    mean = (s_ref[0] + s_ref[1]) / cnt
    hh = h_ref[...]
    mu_ref[...] = _dot_t(mean, wmul_ref[...]) + bmu_ref[...] + _dot_t(hh, wmur_ref[...])
    ls_ref[...] = _dot_t(mean, wlsl_ref[...]) + bls_ref[...] + _dot_t(hh, wlsr_ref[...])

  return pl.pallas_call(
      body,
      grid=(_N // _R,),
      in_specs=[_row_spec(shape3=(_D,)), _row_spec(shape3=(1,)),
                _row_spec(shape2=(_D,)),
                _full_spec((64, _D)), _full_spec((1, 64)), _full_spec((64, _D)),
                _full_spec((64, _D)), _full_spec((1, 64)), _full_spec((64, _D))],
      out_specs=[_row_spec(shape2=(64,)), _row_spec(shape2=(64,))],
      out_shape=[jax.ShapeDtypeStruct((_N, 64), jnp.float32),
                 jax.ShapeDtypeStruct((_N, 64), jnp.float32)],
  )(sum2, cnt2, h, Wmul, bmu, Wmur, Wlsl, bls, Wlsr)


def kernel(x, edge_index, W1l, b1, W1r, Wmul, bmu, Wmur, Wlsl, bls, Wlsr):
  e = edge_index.shape[1]
  e_pad = math.ceil(e / (_LANE * _NW)) * _LANE * _NW
  pad = e_pad - e
  src = jnp.concatenate([edge_index[0], jnp.zeros((pad,), jnp.int32)])
  dst = jnp.concatenate([edge_index[1], jnp.full((pad,), _N, jnp.int32)])
  zrow = jnp.zeros((_APT, _D), jnp.float32)
  zcnt = jnp.zeros((_CPT,), jnp.float32)

  agg_cnt = _make_sc_agg(e_pad, with_cnt=True)
  agg = _make_sc_agg(e_pad, with_cnt=False)

  sum_flat, cnt_flat = agg_cnt(x, src, dst, zrow, zcnt)
  sum2 = sum_flat.reshape(_NC, _ACC_ROWS, _D)[:, :_N]
  cnt2 = cnt_flat.reshape(_NC, _CAC)[:, :_N].reshape(_NC, _N, 1)
  h = _tc_layer1(sum2, cnt2, x, W1l, b1.reshape(1, _D), W1r)

  (sumh_flat,) = agg(h, src, dst, zrow)
  sumh2 = sumh_flat.reshape(_NC, _ACC_ROWS, _D)[:, :_N]
  mu, logstd = _tc_layer23(sumh2, cnt2, h, Wmul, bmu.reshape(1, 64), Wmur,
                           Wlsl, bls.reshape(1, 64), Wlsr)
  return (mu, logstd)


# speedup vs baseline: 1.2628x; 1.0004x over previous
import math

import jax
import jax.numpy as jnp
from jax import lax
from jax.experimental import pallas as pl
from jax.experimental.pallas import tpu as pltpu
from jax.experimental.pallas import tpu_sc as plsc

_N = 10000
_D = 128
_LANE = 128
_NC = 2
_NS = 16
_NW = _NC * _NS

_ACC_ROWS = 10112
_APT = _ACC_ROWS // _NS
_CAC = 10240
_CPT = _CAC // _NS


def _make_sc_agg(e_pad, with_cnt):
  iters_per_tile = e_pad // (_NW * _LANE)
  mesh = plsc.VectorSubcoreMesh(core_axis_name="c", subcore_axis_name="s")

  out_type = [jax.ShapeDtypeStruct((_NC * _ACC_ROWS, _D), jnp.float32)]
  scratch = [
      pltpu.VMEM_SHARED((_ACC_ROWS, _D), jnp.float32),
      pltpu.VMEM((_LANE,), jnp.int32),
      pltpu.VMEM((_LANE,), jnp.int32),
      pltpu.VMEM((_LANE, _D), jnp.float32),
      pltpu.SemaphoreType.DMA,
  ]
  if with_cnt:
    out_type.append(jax.ShapeDtypeStruct((_NC * _CAC,), jnp.float32))
    scratch += [
        pltpu.VMEM_SHARED((_CAC,), jnp.float32),
        pltpu.VMEM((_LANE,), jnp.float32),
        pltpu.VMEM((_CPT,), jnp.float32),
    ]

  def body(*refs):
    if with_cnt:
      (table, srcp, dstp, zrow, zcnt,
       out, cntout, acc, srcv, dstv, rows, gsem, cacc, ones_v, cbuf) = refs
    else:
      (table, srcp, dstp, zrow,
       out, acc, srcv, dstv, rows, gsem) = refs
    c = lax.axis_index("c")
    s = lax.axis_index("s")
    r0 = s * _APT
    pltpu.sync_copy(zrow, acc.at[pl.ds(r0, _APT)])
    if with_cnt:
      c0 = s * _CPT
      for j in range(_LANE // 16):
        ones_v[pl.ds(j * 16, 16)] = jnp.ones((16,), jnp.float32)
      pltpu.sync_copy(zcnt, cbuf)
      pltpu.sync_copy(cbuf, cacc.at[pl.ds(c0, _CPT)])
    plsc.subcore_barrier()

    base = (c * _NS + s) * iters_per_tile * _LANE

    def step(i, carry):
      off = base + i * _LANE
      pltpu.sync_copy(srcp.at[pl.ds(off, _LANE)], srcv)
      pltpu.sync_copy(dstp.at[pl.ds(off, _LANE)], dstv)
      pltpu.async_copy(table.at[srcv], rows, gsem).wait()
      pltpu.sync_copy(rows, acc.at[dstv], add=True)
      if with_cnt:
        pltpu.sync_copy(ones_v, cacc.at[dstv], add=True)
      return carry

    lax.fori_loop(0, iters_per_tile, step, 0)
    plsc.subcore_barrier()
    pltpu.sync_copy(acc.at[pl.ds(r0, _APT)], out.at[pl.ds(c * _ACC_ROWS + r0, _APT)])
    if with_cnt:
      pltpu.sync_copy(cacc.at[pl.ds(c0, _CPT)], cbuf)
      pltpu.sync_copy(cbuf, cntout.at[pl.ds(c * _CAC + c0, _CPT)])
    plsc.subcore_barrier()

  return pl.kernel(body, out_type=out_type, mesh=mesh, scratch_types=scratch)


def _dot_t(a, w):
  return lax.dot_general(a, w, (((1,), (1,)), ((), ())),
                         precision=lax.Precision.HIGHEST,
                         preferred_element_type=jnp.float32)


_R = 2000


def _row_spec(shape3=None, shape2=None):
  if shape3 is not None:
    return pl.BlockSpec((2, _R) + shape3, lambda i: (0, i, 0))
  return pl.BlockSpec((_R,) + shape2, lambda i: (i, 0))


def _full_spec(shape):
  nd = len(shape)
  return pl.BlockSpec(shape, lambda i, _n=nd: (0,) * _n)


def _tc_layer1(sum2, cnt2, x, W1l, b1, W1r):
  def body(s_ref, c_ref, x_ref, wl_ref, b_ref, wr_ref, h_ref):
    cnt = jnp.maximum(c_ref[0] + c_ref[1], 1.0)
    mean = (s_ref[0] + s_ref[1]) / cnt
    h = _dot_t(mean, wl_ref[...]) + b_ref[...] + _dot_t(x_ref[...], wr_ref[...])
    h_ref[...] = jnp.maximum(h, 0.0)

  return pl.pallas_call(
      body,
      grid=(_N // _R,),
      in_specs=[_row_spec(shape3=(_D,)), _row_spec(shape3=(1,)),
                _row_spec(shape2=(_D,)), _full_spec((_D, _D)),
                _full_spec((1, _D)), _full_spec((_D, _D))],
      out_specs=_row_spec(shape2=(_D,)),
      out_shape=jax.ShapeDtypeStruct((_N, _D), jnp.float32),
  )(sum2, cnt2, x, W1l, b1, W1r)


def _tc_layer23(sum2, cnt2, h, Wmul, bmu, Wmur, Wlsl, bls, Wlsr):
  def body(s_ref, c_ref, h_ref, wmul_ref, bmu_ref, wmur_ref,
           wlsl_ref, bls_ref, wlsr_ref, mu_ref, ls_ref):
    cnt = jnp.maximum(c_ref[0] + c_ref[1], 1.0)
    mean = (s_ref[0] + s_ref[1]) / cnt
    hh = h_ref[...]
    mu_ref[...] = _dot_t(mean, wmul_ref[...]) + bmu_ref[...] + _dot_t(hh, wmur_ref[...])
    ls_ref[...] = _dot_t(mean, wlsl_ref[...]) + bls_ref[...] + _dot_t(hh, wlsr_ref[...])

  return pl.pallas_call(
      body,
      grid=(_N // _R,),
      in_specs=[_row_spec(shape3=(_D,)), _row_spec(shape3=(1,)),
                _row_spec(shape2=(_D,)),
                _full_spec((64, _D)), _full_spec((1, 64)), _full_spec((64, _D)),
                _full_spec((64, _D)), _full_spec((1, 64)), _full_spec((64, _D))],
      out_specs=[_row_spec(shape2=(64,)), _row_spec(shape2=(64,))],
      out_shape=[jax.ShapeDtypeStruct((_N, 64), jnp.float32),
                 jax.ShapeDtypeStruct((_N, 64), jnp.float32)],
  )(sum2, cnt2, h, Wmul, bmu, Wmur, Wlsl, bls, Wlsr)


def kernel(x, edge_index, W1l, b1, W1r, Wmul, bmu, Wmur, Wlsl, bls, Wlsr):
  e = edge_index.shape[1]
  e_pad = math.ceil(e / (_LANE * _NW)) * _LANE * _NW
  pad = e_pad - e
  src = jnp.concatenate([edge_index[0], jnp.zeros((pad,), jnp.int32)])
  dst = jnp.concatenate([edge_index[1], jnp.full((pad,), _N, jnp.int32)])
  zrow = jnp.zeros((_APT, _D), jnp.float32)
  zcnt = jnp.zeros((_CPT,), jnp.float32)

  agg_cnt = _make_sc_agg(e_pad, with_cnt=True)
  agg = _make_sc_agg(e_pad, with_cnt=False)

  sum_flat, cnt_flat = agg_cnt(x, src, dst, zrow, zcnt)
  sum2 = sum_flat.reshape(_NC, _ACC_ROWS, _D)[:, :_N]
  cnt2 = cnt_flat.reshape(_NC, _CAC)[:, :_N].reshape(_NC, _N, 1)
  h = _tc_layer1(sum2, cnt2, x, W1l, b1.reshape(1, _D), W1r)

  (sumh_flat,) = agg(h, src, dst, zrow)
  sumh2 = sumh_flat.reshape(_NC, _ACC_ROWS, _D)[:, :_N]
  mu, logstd = _tc_layer23(sumh2, cnt2, h, Wmul, bmu.reshape(1, 64), Wmur,
                           Wlsl, bls.reshape(1, 64), Wlsr)
  return (mu, logstd)
